# baseline (speedup 1.0000x reference)
import jax
import jax.numpy as jnp
from jax.experimental import pallas as pl
import jax.experimental.pallas.tpu as pltpu

TOKENS = 8192
D_IN = 2048
D_OUT = 2048
BM = 1024
NC = 512
NM = TOKENS // BM
NN = D_OUT // NC


def _matmul_kernel(x_ref, w_hbm_ref, o_ref, w_bf16_ref, stage_ref, sem_ref):
    n = pl.program_id(1)
    slot = jax.lax.rem(n, 2)

    @pl.when(pl.program_id(0) == 0)
    def _():
        @pl.when(n == 0)
        def _():
            for c in (0, 1):
                pltpu.make_async_copy(
                    w_hbm_ref.at[pl.ds(c * NC, NC), :],
                    stage_ref.at[c],
                    sem_ref.at[c],
                ).start()

        pltpu.make_async_copy(
            w_hbm_ref.at[pl.ds(n * NC, NC), :],
            stage_ref.at[slot],
            sem_ref.at[slot],
        ).wait()
        w_bf16_ref[n] = stage_ref[slot].astype(jnp.bfloat16)

        @pl.when(n + 2 < NN)
        def _():
            pltpu.make_async_copy(
                w_hbm_ref.at[pl.ds((n + 2) * NC, NC), :],
                stage_ref.at[slot],
                sem_ref.at[slot],
            ).start()

    x_bf16 = x_ref[...].astype(jnp.bfloat16)
    o_ref[...] = jax.lax.dot_general(
        x_bf16,
        w_bf16_ref[n],
        dimension_numbers=(((1,), (1,)), ((), ())),
        preferred_element_type=jnp.float32,
    )


@jax.jit
def kernel(x, W):
    return pl.pallas_call(
        _matmul_kernel,
        grid=(NM, NN),
        in_specs=[
            pl.BlockSpec((BM, D_IN), lambda m, n: (m, 0)),
            pl.BlockSpec(memory_space=pl.ANY),
        ],
        out_specs=pl.BlockSpec((BM, NC), lambda m, n: (m, n)),
        out_shape=jax.ShapeDtypeStruct((TOKENS, D_OUT), jnp.float32),
        scratch_shapes=[
            pltpu.VMEM((NN, NC, D_IN), jnp.bfloat16),
            pltpu.VMEM((2, NC, D_IN), jnp.float32),
            pltpu.SemaphoreType.DMA((2,)),
        ],
        compiler_params=pltpu.CompilerParams(
            vmem_limit_bytes=62 * 1024 * 1024,
        ),
    )(x, W)


# speedup vs baseline: 1.2690x; 1.2690x over previous
import jax
import jax.numpy as jnp
from jax.experimental import pallas as pl
import jax.experimental.pallas.tpu as pltpu

TOKENS = 8192
D_IN = 2048
D_OUT = 2048
BM = 1024


def _matmul_kernel(x_ref, w_ref, o_ref):
    o_ref[...] = jax.lax.dot_general(
        x_ref[...],
        w_ref[...],
        dimension_numbers=(((1,), (1,)), ((), ())),
        precision=jax.lax.Precision.DEFAULT,
        preferred_element_type=jnp.float32,
    )


@jax.jit
def kernel(x, W):
    return pl.pallas_call(
        _matmul_kernel,
        grid=(TOKENS // BM,),
        in_specs=[
            pl.BlockSpec((BM, D_IN), lambda i: (i, 0)),
            pl.BlockSpec((D_OUT, D_IN), lambda i: (0, 0)),
        ],
        out_specs=pl.BlockSpec((BM, D_OUT), lambda i: (i, 0)),
        out_shape=jax.ShapeDtypeStruct((TOKENS, D_OUT), jnp.float32),
        compiler_params=pltpu.CompilerParams(
            vmem_limit_bytes=62 * 1024 * 1024,
        ),
    )(x, W)
